# baseline (speedup 1.0000x reference)
import functools

import jax
import jax.numpy as jnp
import numpy as np
from jax import lax
from jax.experimental import pallas as pl
from jax.experimental.pallas import tpu as pltpu
from jax.experimental.pallas import tpu_sc as plsc

N = 10000
E = 320000
D = 128
DW = 160
DH = DW // 2
NROW = 10240
NC = 2
NS = 16
CB = 40
CHUNKS = 500
P = 5
RPT = NROW // NS

_KEEP_CACHE = []


def _np_threefry2x32(k1, k2, x1, x2):
    rot = (13, 15, 26, 6, 17, 29, 16, 24)

    def rotl(x, d):
        return ((x << np.uint32(d)) | (x >> np.uint32(32 - d))).astype(np.uint32)

    ks = [np.uint32(k1), np.uint32(k2), np.uint32(k1 ^ k2 ^ 0x1BD11BDA)]
    x = [x1.astype(np.uint32) + ks[0], x2.astype(np.uint32) + ks[1]]
    with np.errstate(over="ignore"):
        for i in range(5):
            for r in (rot[:4], rot[4:])[i % 2]:
                x[0] = (x[0] + x[1]).astype(np.uint32)
                x[1] = x[0] ^ rotl(x[1], r)
            x[0] = (x[0] + ks[(i + 1) % 3]).astype(np.uint32)
            x[1] = (x[1] + ks[(i + 2) % 3] + np.uint32(i + 1)).astype(np.uint32)
    return x


def _keep_mask():
    if not _KEEP_CACHE:
        b1, b2 = _np_threefry2x32(0, 42, np.zeros(E, np.uint32),
                                  np.arange(E, dtype=np.uint32))
        bits = b1 ^ b2
        u = ((bits >> np.uint32(9)) | np.uint32(0x3F800000)).view(np.float32)
        u = np.maximum(np.float32(0.0), u - np.float32(1.0))
        _KEEP_CACHE.append(u >= np.float32(0.3))
    return _KEEP_CACHE[0]


def _seg_sum(t_lo, t_hi, srcv, dstv, z):
    mesh = plsc.VectorSubcoreMesh(
        core_axis_name="c", subcore_axis_name="s", num_cores=NC, num_subcores=NS
    )
    scratch = (
        [pltpu.VMEM((CHUNKS, CB), jnp.int32)] * 2
        + [pltpu.VMEM((CB, DH), jnp.float32)] * P
        + [pltpu.VMEM_SHARED((NROW, DH), jnp.float32)]
        + [pltpu.SemaphoreType.DMA] * P
    )

    @functools.partial(
        pl.kernel,
        mesh=mesh,
        out_type=[jax.ShapeDtypeStruct((NROW, DH), jnp.float32),
                  jax.ShapeDtypeStruct((NROW, DH), jnp.float32)],
        scratch_types=scratch,
        compiler_params=pltpu.CompilerParams(use_tc_tiling_on_sc=False),
    )
    def body(tlo_h, thi_h, src_h, dst_h, z_h, outlo_h, outhi_h,
             src_v, dst_v, b0, b1, b2, b3, b4, acc, s0, s1, s2, s3, s4):
        bufs = [b0, b1, b2, b3, b4]
        sems = [s0, s1, s2, s3, s4]
        cid = lax.axis_index("c")
        sid = lax.axis_index("s")
        pltpu.sync_copy(src_h.at[sid], src_v)
        pltpu.sync_copy(dst_h.at[sid], dst_v)
        pltpu.sync_copy(z_h, b0)
        r0 = sid * RPT
        for k in range(RPT // CB):
            pltpu.sync_copy(b0, acc.at[pl.ds(r0 + k * CB, CB)])
        plsc.subcore_barrier()

        def pipeline(table_h):
            for b in range(P):
                pltpu.async_copy(table_h.at[src_v.at[b]], bufs[b], sems[b])

            @pl.loop(0, CHUNKS - P, step=P)
            def _(j0):
                for b in range(P):
                    pltpu.make_async_copy(
                        table_h.at[src_v.at[b]], bufs[b], sems[b]).wait()
                    pltpu.sync_copy(bufs[b], acc.at[dst_v.at[j0 + b]],
                                    add=True)
                    pltpu.async_copy(
                        table_h.at[src_v.at[j0 + b + P]], bufs[b], sems[b])

            for b in range(P):
                pltpu.make_async_copy(
                    table_h.at[src_v.at[b]], bufs[b], sems[b]).wait()
                pltpu.sync_copy(
                    bufs[b], acc.at[dst_v.at[(CHUNKS - P) + b]], add=True)
            plsc.subcore_barrier()

        @pl.when(cid == 0)
        def _():
            pipeline(tlo_h)
            pltpu.sync_copy(acc.at[pl.ds(r0, RPT)], outlo_h.at[pl.ds(r0, RPT)])

        @pl.when(cid == 1)
        def _():
            pipeline(thi_h)
            pltpu.sync_copy(acc.at[pl.ds(r0, RPT)], outhi_h.at[pl.ds(r0, RPT)])

    return body(t_lo, t_hi, srcv, dstv, z)


def _dense1(acc_lo, acc_hi, x, w1_lT, w1_rT, w_skipT, b1_l, b_skip):
    RB = 1000

    def body(alo_ref, ahi_ref, x_ref, wl_ref, wr_ref, ws_ref,
             bl_ref, bs_ref, helo_ref, hehi_ref):
        a = jnp.concatenate([alo_ref[...], ahi_ref[...]], axis=1)
        mean = a[:, :D] / jnp.maximum(a[:, D:D + 1], 1.0)
        xb = x_ref[...]
        out1 = (jnp.dot(mean, wl_ref[...], preferred_element_type=jnp.float32)
                + bl_ref[...]
                + jnp.dot(xb, wr_ref[...], preferred_element_type=jnp.float32))
        nrm = jnp.sqrt(jnp.sum(out1 * out1, axis=1, keepdims=True))
        h = out1 / jnp.maximum(nrm, 1e-12)
        h = jnp.tanh(
            h + jnp.dot(xb, ws_ref[...], preferred_element_type=jnp.float32)
            + bs_ref[...])
        helo_ref[...] = h[:, :DH]
        hehi_ref[...] = jnp.concatenate(
            [h[:, DH:], jnp.ones((RB, 1), jnp.float32),
             jnp.zeros((RB, DW - D - 1), jnp.float32)], axis=1)

    return pl.pallas_call(
        body,
        grid=(N // RB,),
        in_specs=[
            pl.BlockSpec((RB, DH), lambda i: (i, 0)),
            pl.BlockSpec((RB, DH), lambda i: (i, 0)),
            pl.BlockSpec((RB, D), lambda i: (i, 0)),
            pl.BlockSpec((D, D), lambda i: (0, 0)),
            pl.BlockSpec((D, D), lambda i: (0, 0)),
            pl.BlockSpec((D, D), lambda i: (0, 0)),
            pl.BlockSpec((1, D), lambda i: (0, 0)),
            pl.BlockSpec((1, D), lambda i: (0, 0)),
        ],
        out_specs=[pl.BlockSpec((RB, DH), lambda i: (i, 0)),
                   pl.BlockSpec((RB, DH), lambda i: (i, 0))],
        out_shape=[jax.ShapeDtypeStruct((N, DH), jnp.float32),
                   jax.ShapeDtypeStruct((N, DH), jnp.float32)],
    )(acc_lo, acc_hi, x, w1_lT, w1_rT, w_skipT, b1_l, b_skip)


def _dense2(acc_lo, acc_hi, he_lo, he_hi, w2_lT, w2_rT, w_linT, b2_l, b_lin):
    RB = 2000

    def body(alo_ref, ahi_ref, hlo_ref, hhi_ref, wl_ref, wr_ref, wo_ref,
             bl_ref, bo_ref, out_ref):
        a = jnp.concatenate([alo_ref[...], ahi_ref[...]], axis=1)
        mean = a[:, :D] / jnp.maximum(a[:, D:D + 1], 1.0)
        hb = jnp.concatenate([hlo_ref[...], hhi_ref[:, :D - DH]], axis=1)
        out2 = (jnp.dot(mean, wl_ref[...], preferred_element_type=jnp.float32)
                + bl_ref[...]
                + jnp.dot(hb, wr_ref[...], preferred_element_type=jnp.float32))
        nrm = jnp.sqrt(jnp.sum(out2 * out2, axis=1, keepdims=True))
        h2 = jnp.tanh(out2 / jnp.maximum(nrm, 1e-12))
        out_ref[...] = (
            jnp.dot(h2, wo_ref[...], preferred_element_type=jnp.float32)
            + bo_ref[...])

    return pl.pallas_call(
        body,
        grid=(N // RB,),
        in_specs=[
            pl.BlockSpec((RB, DH), lambda i: (i, 0)),
            pl.BlockSpec((RB, DH), lambda i: (i, 0)),
            pl.BlockSpec((RB, DH), lambda i: (i, 0)),
            pl.BlockSpec((RB, DH), lambda i: (i, 0)),
            pl.BlockSpec((D, D), lambda i: (0, 0)),
            pl.BlockSpec((D, D), lambda i: (0, 0)),
            pl.BlockSpec((D, D), lambda i: (0, 0)),
            pl.BlockSpec((1, D), lambda i: (0, 0)),
            pl.BlockSpec((1, D), lambda i: (0, 0)),
        ],
        out_specs=pl.BlockSpec((RB, D), lambda i: (i, 0)),
        out_shape=jax.ShapeDtypeStruct((N, D), jnp.float32),
    )(acc_lo, acc_hi, he_lo, he_hi, w2_lT, w2_rT, w_linT, b2_l, b_lin)


def kernel(x, edge_index, w1_l, b1_l, w1_r, w_skip, b_skip,
           w2_l, b2_l, w2_r, w_lin, b_lin):
    src = edge_index[0].astype(jnp.int32)
    dst = edge_index[1].astype(jnp.int32)
    srcv = src.reshape(NS, CHUNKS, CB)
    dst1v = dst.reshape(NS, CHUNKS, CB)
    keep = jnp.asarray(_keep_mask())
    trash_e = jnp.asarray(N + (np.arange(E) % (NROW - N)), jnp.int32)
    dst2v = jnp.where(keep, dst, trash_e).reshape(NS, CHUNKS, CB)

    xe_lo = x[:, :DH]
    xe_hi = jnp.concatenate(
        [x[:, DH:], jnp.ones((N, 1), jnp.float32),
         jnp.zeros((N, DW - D - 1), jnp.float32)], axis=1)
    z = jnp.zeros((CB, DH), jnp.float32)

    acc1_lo, acc1_hi = _seg_sum(xe_lo, xe_hi, srcv, dst1v, z)
    he_lo, he_hi = _dense1(acc1_lo, acc1_hi, x, w1_l.T, w1_r.T, w_skip.T,
                           b1_l[None, :], b_skip[None, :])
    acc2_lo, acc2_hi = _seg_sum(he_lo, he_hi, srcv, dst2v, z)
    return _dense2(acc2_lo, acc2_hi, he_lo, he_hi,
                   w2_l.T, w2_r.T, w_lin.T, b2_l[None, :], b_lin[None, :])


# speedup vs baseline: 1.0580x; 1.0580x over previous
import functools

import jax
import jax.numpy as jnp
import numpy as np
from jax import lax
from jax.experimental import pallas as pl
from jax.experimental.pallas import tpu as pltpu
from jax.experimental.pallas import tpu_sc as plsc

N = 10000
E = 320000
D = 128
DW = 160
DH = DW // 2
NROW = 10240
NC = 2
NS = 16
CB = 80
CHUNKS = 250
P = 5
RPT = NROW // NS

_KEEP_CACHE = []


def _np_threefry2x32(k1, k2, x1, x2):
    rot = (13, 15, 26, 6, 17, 29, 16, 24)

    def rotl(x, d):
        return ((x << np.uint32(d)) | (x >> np.uint32(32 - d))).astype(np.uint32)

    ks = [np.uint32(k1), np.uint32(k2), np.uint32(k1 ^ k2 ^ 0x1BD11BDA)]
    x = [x1.astype(np.uint32) + ks[0], x2.astype(np.uint32) + ks[1]]
    with np.errstate(over="ignore"):
        for i in range(5):
            for r in (rot[:4], rot[4:])[i % 2]:
                x[0] = (x[0] + x[1]).astype(np.uint32)
                x[1] = x[0] ^ rotl(x[1], r)
            x[0] = (x[0] + ks[(i + 1) % 3]).astype(np.uint32)
            x[1] = (x[1] + ks[(i + 2) % 3] + np.uint32(i + 1)).astype(np.uint32)
    return x


def _keep_mask():
    if not _KEEP_CACHE:
        b1, b2 = _np_threefry2x32(0, 42, np.zeros(E, np.uint32),
                                  np.arange(E, dtype=np.uint32))
        bits = b1 ^ b2
        u = ((bits >> np.uint32(9)) | np.uint32(0x3F800000)).view(np.float32)
        u = np.maximum(np.float32(0.0), u - np.float32(1.0))
        _KEEP_CACHE.append(u >= np.float32(0.3))
    return _KEEP_CACHE[0]


def _seg_sum(t_lo, t_hi, eiv, dst2v, z, use_dst2):
    mesh = plsc.VectorSubcoreMesh(
        core_axis_name="c", subcore_axis_name="s", num_cores=NC, num_subcores=NS
    )
    scratch = (
        [pltpu.VMEM((CHUNKS, CB), jnp.int32)] * 2
        + [pltpu.VMEM((CB, DH), jnp.float32)] * P
        + [pltpu.VMEM_SHARED((NROW, DH), jnp.float32)]
        + [pltpu.SemaphoreType.DMA] * P
    )

    @functools.partial(
        pl.kernel,
        mesh=mesh,
        out_type=[jax.ShapeDtypeStruct((NROW, DH), jnp.float32),
                  jax.ShapeDtypeStruct((NROW, DH), jnp.float32)],
        scratch_types=scratch,
        compiler_params=pltpu.CompilerParams(use_tc_tiling_on_sc=False),
    )
    def body(ei_h, dst2_h, tlo_h, thi_h, z_h, outlo_h, outhi_h,
             src_v, dst_v, b0, b1, b2, b3, b4, acc, s0, s1, s2, s3, s4):
        bufs = [b0, b1, b2, b3, b4]
        sems = [s0, s1, s2, s3, s4]
        cid = lax.axis_index("c")
        sid = lax.axis_index("s")
        pltpu.sync_copy(ei_h.at[0, sid], src_v)
        if use_dst2:
            pltpu.sync_copy(dst2_h.at[sid], dst_v)
        else:
            pltpu.sync_copy(ei_h.at[1, sid], dst_v)
        pltpu.sync_copy(z_h, b0)
        r0 = sid * RPT
        for k in range(RPT // CB):
            pltpu.sync_copy(b0, acc.at[pl.ds(r0 + k * CB, CB)])
        plsc.subcore_barrier()

        def pipeline(table_h):
            for b in range(P):
                pltpu.async_copy(table_h.at[src_v.at[b]], bufs[b], sems[b])

            @pl.loop(0, CHUNKS - P, step=P)
            def _(j0):
                for b in range(P):
                    pltpu.make_async_copy(
                        table_h.at[src_v.at[b]], bufs[b], sems[b]).wait()
                    pltpu.sync_copy(bufs[b], acc.at[dst_v.at[j0 + b]],
                                    add=True)
                    pltpu.async_copy(
                        table_h.at[src_v.at[j0 + b + P]], bufs[b], sems[b])

            for b in range(P):
                pltpu.make_async_copy(
                    table_h.at[src_v.at[b]], bufs[b], sems[b]).wait()
                pltpu.sync_copy(
                    bufs[b], acc.at[dst_v.at[(CHUNKS - P) + b]], add=True)
            plsc.subcore_barrier()

        @pl.when(cid == 0)
        def _():
            pipeline(tlo_h)
            pltpu.sync_copy(acc.at[pl.ds(r0, RPT)], outlo_h.at[pl.ds(r0, RPT)])

        @pl.when(cid == 1)
        def _():
            pipeline(thi_h)
            pltpu.sync_copy(acc.at[pl.ds(r0, RPT)], outhi_h.at[pl.ds(r0, RPT)])

    return body(eiv, dst2v, t_lo, t_hi, z)


def _dense1(acc_lo, acc_hi, x, w1_lT, w1_rT, w_skipT, b1_l, b_skip):
    RB = 1000

    def body(alo_ref, ahi_ref, x_ref, wl_ref, wr_ref, ws_ref,
             bl_ref, bs_ref, helo_ref, hehi_ref):
        a = jnp.concatenate([alo_ref[...], ahi_ref[...]], axis=1)
        mean = a[:, :D] / jnp.maximum(a[:, D:D + 1], 1.0)
        xb = x_ref[...]
        out1 = (jnp.dot(mean, wl_ref[...], preferred_element_type=jnp.float32)
                + bl_ref[...]
                + jnp.dot(xb, wr_ref[...], preferred_element_type=jnp.float32))
        nrm = jnp.sqrt(jnp.sum(out1 * out1, axis=1, keepdims=True))
        h = out1 / jnp.maximum(nrm, 1e-12)
        h = jnp.tanh(
            h + jnp.dot(xb, ws_ref[...], preferred_element_type=jnp.float32)
            + bs_ref[...])
        helo_ref[...] = h[:, :DH]
        hehi_ref[...] = jnp.concatenate(
            [h[:, DH:], jnp.ones((RB, 1), jnp.float32),
             jnp.zeros((RB, DW - D - 1), jnp.float32)], axis=1)

    return pl.pallas_call(
        body,
        grid=(N // RB,),
        in_specs=[
            pl.BlockSpec((RB, DH), lambda i: (i, 0)),
            pl.BlockSpec((RB, DH), lambda i: (i, 0)),
            pl.BlockSpec((RB, D), lambda i: (i, 0)),
            pl.BlockSpec((D, D), lambda i: (0, 0)),
            pl.BlockSpec((D, D), lambda i: (0, 0)),
            pl.BlockSpec((D, D), lambda i: (0, 0)),
            pl.BlockSpec((1, D), lambda i: (0, 0)),
            pl.BlockSpec((1, D), lambda i: (0, 0)),
        ],
        out_specs=[pl.BlockSpec((RB, DH), lambda i: (i, 0)),
                   pl.BlockSpec((RB, DH), lambda i: (i, 0))],
        out_shape=[jax.ShapeDtypeStruct((N, DH), jnp.float32),
                   jax.ShapeDtypeStruct((N, DH), jnp.float32)],
    )(acc_lo, acc_hi, x, w1_lT, w1_rT, w_skipT, b1_l, b_skip)


def _dense2(acc_lo, acc_hi, he_lo, he_hi, w2_lT, w2_rT, w_linT, b2_l, b_lin):
    RB = 2000

    def body(alo_ref, ahi_ref, hlo_ref, hhi_ref, wl_ref, wr_ref, wo_ref,
             bl_ref, bo_ref, out_ref):
        a = jnp.concatenate([alo_ref[...], ahi_ref[...]], axis=1)
        mean = a[:, :D] / jnp.maximum(a[:, D:D + 1], 1.0)
        hb = jnp.concatenate([hlo_ref[...], hhi_ref[:, :D - DH]], axis=1)
        out2 = (jnp.dot(mean, wl_ref[...], preferred_element_type=jnp.float32)
                + bl_ref[...]
                + jnp.dot(hb, wr_ref[...], preferred_element_type=jnp.float32))
        nrm = jnp.sqrt(jnp.sum(out2 * out2, axis=1, keepdims=True))
        h2 = jnp.tanh(out2 / jnp.maximum(nrm, 1e-12))
        out_ref[...] = (
            jnp.dot(h2, wo_ref[...], preferred_element_type=jnp.float32)
            + bo_ref[...])

    return pl.pallas_call(
        body,
        grid=(N // RB,),
        in_specs=[
            pl.BlockSpec((RB, DH), lambda i: (i, 0)),
            pl.BlockSpec((RB, DH), lambda i: (i, 0)),
            pl.BlockSpec((RB, DH), lambda i: (i, 0)),
            pl.BlockSpec((RB, DH), lambda i: (i, 0)),
            pl.BlockSpec((D, D), lambda i: (0, 0)),
            pl.BlockSpec((D, D), lambda i: (0, 0)),
            pl.BlockSpec((D, D), lambda i: (0, 0)),
            pl.BlockSpec((1, D), lambda i: (0, 0)),
            pl.BlockSpec((1, D), lambda i: (0, 0)),
        ],
        out_specs=pl.BlockSpec((RB, D), lambda i: (i, 0)),
        out_shape=jax.ShapeDtypeStruct((N, D), jnp.float32),
    )(acc_lo, acc_hi, he_lo, he_hi, w2_lT, w2_rT, w_linT, b2_l, b_lin)


def kernel(x, edge_index, w1_l, b1_l, w1_r, w_skip, b_skip,
           w2_l, b2_l, w2_r, w_lin, b_lin):
    eiv = edge_index.astype(jnp.int32).reshape(2, NS, CHUNKS, CB)
    keep = jnp.asarray(_keep_mask())
    trash_e = jnp.asarray(N + (np.arange(E) % (NROW - N)), jnp.int32)
    dst2v = jnp.where(keep, edge_index[1].astype(jnp.int32),
                      trash_e).reshape(NS, CHUNKS, CB)

    xe_lo = x[:, :DH]
    xe_hi = jnp.concatenate(
        [x[:, DH:], jnp.ones((N, 1), jnp.float32),
         jnp.zeros((N, DW - D - 1), jnp.float32)], axis=1)
    z = jnp.zeros((CB, DH), jnp.float32)

    acc1_lo, acc1_hi = _seg_sum(xe_lo, xe_hi, eiv, dst2v, z, False)
    he_lo, he_hi = _dense1(acc1_lo, acc1_hi, x, w1_l.T, w1_r.T, w_skip.T,
                           b1_l[None, :], b_skip[None, :])
    acc2_lo, acc2_hi = _seg_sum(he_lo, he_hi, eiv, dst2v, z, True)
    return _dense2(acc2_lo, acc2_hi, he_lo, he_hi,
                   w2_l.T, w2_r.T, w_lin.T, b2_l[None, :], b_lin[None, :])


# speedup vs baseline: 1.0769x; 1.0178x over previous
import functools

import jax
import jax.numpy as jnp
import numpy as np
from jax import lax
from jax.experimental import pallas as pl
from jax.experimental.pallas import tpu as pltpu
from jax.experimental.pallas import tpu_sc as plsc

N = 10000
E = 320000
D = 128
DW = 160
DH = DW // 2
NROW = 10240
NC = 2
NS = 16
CB = 80
CHUNKS = 250
P = 5
RPT = NROW // NS

_KEEP_CACHE = []


def _np_threefry2x32(k1, k2, x1, x2):
    rot = (13, 15, 26, 6, 17, 29, 16, 24)

    def rotl(x, d):
        return ((x << np.uint32(d)) | (x >> np.uint32(32 - d))).astype(np.uint32)

    ks = [np.uint32(k1), np.uint32(k2), np.uint32(k1 ^ k2 ^ 0x1BD11BDA)]
    x = [x1.astype(np.uint32) + ks[0], x2.astype(np.uint32) + ks[1]]
    with np.errstate(over="ignore"):
        for i in range(5):
            for r in (rot[:4], rot[4:])[i % 2]:
                x[0] = (x[0] + x[1]).astype(np.uint32)
                x[1] = x[0] ^ rotl(x[1], r)
            x[0] = (x[0] + ks[(i + 1) % 3]).astype(np.uint32)
            x[1] = (x[1] + ks[(i + 2) % 3] + np.uint32(i + 1)).astype(np.uint32)
    return x


def _keep_mask():
    if not _KEEP_CACHE:
        b1, b2 = _np_threefry2x32(0, 42, np.zeros(E, np.uint32),
                                  np.arange(E, dtype=np.uint32))
        bits = b1 ^ b2
        u = ((bits >> np.uint32(9)) | np.uint32(0x3F800000)).view(np.float32)
        u = np.maximum(np.float32(0.0), u - np.float32(1.0))
        _KEEP_CACHE.append(u >= np.float32(0.3))
    return _KEEP_CACHE[0]


def _seg_sum(t_lo, t_hi, srcv, dstv, z):
    mesh = plsc.VectorSubcoreMesh(
        core_axis_name="c", subcore_axis_name="s", num_cores=NC, num_subcores=NS
    )
    scratch = (
        [pltpu.VMEM((CHUNKS, CB), jnp.int32)] * 2
        + [pltpu.VMEM((CB, DH), jnp.float32)] * P
        + [pltpu.VMEM_SHARED((NROW, DH), jnp.float32)]
        + [pltpu.SemaphoreType.DMA] * P
    )

    @functools.partial(
        pl.kernel,
        mesh=mesh,
        out_type=[jax.ShapeDtypeStruct((NROW, DH), jnp.float32),
                  jax.ShapeDtypeStruct((NROW, DH), jnp.float32)],
        scratch_types=scratch,
        compiler_params=pltpu.CompilerParams(use_tc_tiling_on_sc=False),
    )
    def body(tlo_h, thi_h, src_h, dst_h, z_h, outlo_h, outhi_h,
             src_v, dst_v, b0, b1, b2, b3, b4, acc, s0, s1, s2, s3, s4):
        bufs = [b0, b1, b2, b3, b4]
        sems = [s0, s1, s2, s3, s4]
        cid = lax.axis_index("c")
        sid = lax.axis_index("s")
        pltpu.sync_copy(src_h.at[sid], src_v)
        pltpu.sync_copy(dst_h.at[sid], dst_v)
        pltpu.sync_copy(z_h, b0)
        r0 = sid * RPT
        for k in range(RPT // CB):
            pltpu.sync_copy(b0, acc.at[pl.ds(r0 + k * CB, CB)])
        plsc.subcore_barrier()

        def pipeline(table_h):
            for b in range(P):
                pltpu.async_copy(table_h.at[src_v.at[b]], bufs[b], sems[b])

            @pl.loop(0, CHUNKS - P, step=P)
            def _(j0):
                for b in range(P):
                    pltpu.make_async_copy(
                        table_h.at[src_v.at[b]], bufs[b], sems[b]).wait()
                    pltpu.sync_copy(bufs[b], acc.at[dst_v.at[j0 + b]],
                                    add=True)
                    pltpu.async_copy(
                        table_h.at[src_v.at[j0 + b + P]], bufs[b], sems[b])

            for b in range(P):
                pltpu.make_async_copy(
                    table_h.at[src_v.at[b]], bufs[b], sems[b]).wait()
                pltpu.sync_copy(
                    bufs[b], acc.at[dst_v.at[(CHUNKS - P) + b]], add=True)
            plsc.subcore_barrier()

        @pl.when(cid == 0)
        def _():
            pipeline(tlo_h)
            pltpu.sync_copy(acc.at[pl.ds(r0, RPT)], outlo_h.at[pl.ds(r0, RPT)])

        @pl.when(cid == 1)
        def _():
            pipeline(thi_h)
            pltpu.sync_copy(acc.at[pl.ds(r0, RPT)], outhi_h.at[pl.ds(r0, RPT)])

    return body(t_lo, t_hi, srcv, dstv, z)


def _dense1(acc_lo, acc_hi, x, w1_lT, w1_rT, w_skipT, b1_l, b_skip):
    RB = 1000

    def body(alo_ref, ahi_ref, x_ref, wl_ref, wr_ref, ws_ref,
             bl_ref, bs_ref, helo_ref, hehi_ref):
        a = jnp.concatenate([alo_ref[...], ahi_ref[...]], axis=1)
        mean = a[:, :D] / jnp.maximum(a[:, D:D + 1], 1.0)
        xb = x_ref[...]
        out1 = (jnp.dot(mean, wl_ref[...], preferred_element_type=jnp.float32)
                + bl_ref[...]
                + jnp.dot(xb, wr_ref[...], preferred_element_type=jnp.float32))
        nrm = jnp.sqrt(jnp.sum(out1 * out1, axis=1, keepdims=True))
        h = out1 / jnp.maximum(nrm, 1e-12)
        h = jnp.tanh(
            h + jnp.dot(xb, ws_ref[...], preferred_element_type=jnp.float32)
            + bs_ref[...])
        helo_ref[...] = h[:, :DH]
        hehi_ref[...] = jnp.concatenate(
            [h[:, DH:], jnp.ones((RB, 1), jnp.float32),
             jnp.zeros((RB, DW - D - 1), jnp.float32)], axis=1)

    return pl.pallas_call(
        body,
        grid=(N // RB,),
        in_specs=[
            pl.BlockSpec((RB, DH), lambda i: (i, 0)),
            pl.BlockSpec((RB, DH), lambda i: (i, 0)),
            pl.BlockSpec((RB, D), lambda i: (i, 0)),
            pl.BlockSpec((D, D), lambda i: (0, 0)),
            pl.BlockSpec((D, D), lambda i: (0, 0)),
            pl.BlockSpec((D, D), lambda i: (0, 0)),
            pl.BlockSpec((1, D), lambda i: (0, 0)),
            pl.BlockSpec((1, D), lambda i: (0, 0)),
        ],
        out_specs=[pl.BlockSpec((RB, DH), lambda i: (i, 0)),
                   pl.BlockSpec((RB, DH), lambda i: (i, 0))],
        out_shape=[jax.ShapeDtypeStruct((N, DH), jnp.float32),
                   jax.ShapeDtypeStruct((N, DH), jnp.float32)],
    )(acc_lo, acc_hi, x, w1_lT, w1_rT, w_skipT, b1_l, b_skip)


def _dense2(acc_lo, acc_hi, he_lo, he_hi, w2_lT, w2_rT, w_linT, b2_l, b_lin):
    RB = 2000

    def body(alo_ref, ahi_ref, hlo_ref, hhi_ref, wl_ref, wr_ref, wo_ref,
             bl_ref, bo_ref, out_ref):
        a = jnp.concatenate([alo_ref[...], ahi_ref[...]], axis=1)
        mean = a[:, :D] / jnp.maximum(a[:, D:D + 1], 1.0)
        hb = jnp.concatenate([hlo_ref[...], hhi_ref[:, :D - DH]], axis=1)
        out2 = (jnp.dot(mean, wl_ref[...], preferred_element_type=jnp.float32)
                + bl_ref[...]
                + jnp.dot(hb, wr_ref[...], preferred_element_type=jnp.float32))
        nrm = jnp.sqrt(jnp.sum(out2 * out2, axis=1, keepdims=True))
        h2 = jnp.tanh(out2 / jnp.maximum(nrm, 1e-12))
        out_ref[...] = (
            jnp.dot(h2, wo_ref[...], preferred_element_type=jnp.float32)
            + bo_ref[...])

    return pl.pallas_call(
        body,
        grid=(N // RB,),
        in_specs=[
            pl.BlockSpec((RB, DH), lambda i: (i, 0)),
            pl.BlockSpec((RB, DH), lambda i: (i, 0)),
            pl.BlockSpec((RB, DH), lambda i: (i, 0)),
            pl.BlockSpec((RB, DH), lambda i: (i, 0)),
            pl.BlockSpec((D, D), lambda i: (0, 0)),
            pl.BlockSpec((D, D), lambda i: (0, 0)),
            pl.BlockSpec((D, D), lambda i: (0, 0)),
            pl.BlockSpec((1, D), lambda i: (0, 0)),
            pl.BlockSpec((1, D), lambda i: (0, 0)),
        ],
        out_specs=pl.BlockSpec((RB, D), lambda i: (i, 0)),
        out_shape=jax.ShapeDtypeStruct((N, D), jnp.float32),
    )(acc_lo, acc_hi, he_lo, he_hi, w2_lT, w2_rT, w_linT, b2_l, b_lin)


def kernel(x, edge_index, w1_l, b1_l, w1_r, w_skip, b_skip,
           w2_l, b2_l, w2_r, w_lin, b_lin):
    src = edge_index[0].astype(jnp.int32)
    dst = edge_index[1].astype(jnp.int32)
    srcv = src.reshape(NS, CHUNKS, CB)
    dst1v = dst.reshape(NS, CHUNKS, CB)
    keep = jnp.asarray(_keep_mask())
    trash_e = jnp.asarray(N + (np.arange(E) % (NROW - N)), jnp.int32)
    dst2v = jnp.where(keep, dst, trash_e).reshape(NS, CHUNKS, CB)

    xe_lo = x[:, :DH]
    xe_hi = jnp.concatenate(
        [x[:, DH:], jnp.ones((N, 1), jnp.float32),
         jnp.zeros((N, DW - D - 1), jnp.float32)], axis=1)
    z = jnp.zeros((CB, DH), jnp.float32)

    acc1_lo, acc1_hi = _seg_sum(xe_lo, xe_hi, srcv, dst1v, z)
    he_lo, he_hi = _dense1(acc1_lo, acc1_hi, x, w1_l.T, w1_r.T, w_skip.T,
                           b1_l[None, :], b_skip[None, :])
    acc2_lo, acc2_hi = _seg_sum(he_lo, he_hi, srcv, dst2v, z)
    return _dense2(acc2_lo, acc2_hi, he_lo, he_hi,
                   w2_l.T, w2_r.T, w_lin.T, b2_l[None, :], b_lin[None, :])


# speedup vs baseline: 1.0994x; 1.0209x over previous
import functools

import jax
import jax.numpy as jnp
import numpy as np
from jax import lax
from jax.experimental import pallas as pl
from jax.experimental.pallas import tpu as pltpu
from jax.experimental.pallas import tpu_sc as plsc

N = 10000
E = 320000
D = 128
DH = 64
CW = 16
NROW = 10240
NC = 2
NS = 16
CB = 80
CHUNKS = 250
P = 5
RPT = NROW // NS

_KEEP_CACHE = []


def _np_threefry2x32(k1, k2, x1, x2):
    rot = (13, 15, 26, 6, 17, 29, 16, 24)

    def rotl(x, d):
        return ((x << np.uint32(d)) | (x >> np.uint32(32 - d))).astype(np.uint32)

    ks = [np.uint32(k1), np.uint32(k2), np.uint32(k1 ^ k2 ^ 0x1BD11BDA)]
    x = [x1.astype(np.uint32) + ks[0], x2.astype(np.uint32) + ks[1]]
    with np.errstate(over="ignore"):
        for i in range(5):
            for r in (rot[:4], rot[4:])[i % 2]:
                x[0] = (x[0] + x[1]).astype(np.uint32)
                x[1] = x[0] ^ rotl(x[1], r)
            x[0] = (x[0] + ks[(i + 1) % 3]).astype(np.uint32)
            x[1] = (x[1] + ks[(i + 2) % 3] + np.uint32(i + 1)).astype(np.uint32)
    return x


def _keep_mask():
    if not _KEEP_CACHE:
        b1, b2 = _np_threefry2x32(0, 42, np.zeros(E, np.uint32),
                                  np.arange(E, dtype=np.uint32))
        bits = b1 ^ b2
        u = ((bits >> np.uint32(9)) | np.uint32(0x3F800000)).view(np.float32)
        u = np.maximum(np.float32(0.0), u - np.float32(1.0))
        _KEEP_CACHE.append(u >= np.float32(0.3))
    return _KEEP_CACHE[0]


def _seg_sum(t_lo, t_hi, srcv, dstv, z, ones16):
    mesh = plsc.VectorSubcoreMesh(
        core_axis_name="c", subcore_axis_name="s", num_cores=NC, num_subcores=NS
    )
    scratch = (
        [pltpu.VMEM((CHUNKS, CB), jnp.int32)] * 2
        + [pltpu.VMEM((CB, DH), jnp.float32)] * P
        + [pltpu.VMEM((CB, CW), jnp.float32)]
        + [pltpu.VMEM_SHARED((NROW, DH), jnp.float32)]
        + [pltpu.VMEM_SHARED((NROW, CW), jnp.float32)]
        + [pltpu.SemaphoreType.DMA] * P
    )

    @functools.partial(
        pl.kernel,
        mesh=mesh,
        out_type=[jax.ShapeDtypeStruct((NROW, DH), jnp.float32),
                  jax.ShapeDtypeStruct((NROW, DH), jnp.float32),
                  jax.ShapeDtypeStruct((NROW, CW), jnp.float32)],
        scratch_types=scratch,
        compiler_params=pltpu.CompilerParams(use_tc_tiling_on_sc=False),
    )
    def body(tlo_h, thi_h, src_h, dst_h, z_h, ones_h,
             outlo_h, outhi_h, cnt_h,
             src_v, dst_v, b0, b1, b2, b3, b4, ones_v, acc, cnt_sh,
             s0, s1, s2, s3, s4):
        bufs = [b0, b1, b2, b3, b4]
        sems = [s0, s1, s2, s3, s4]
        cid = lax.axis_index("c")
        sid = lax.axis_index("s")
        pltpu.sync_copy(src_h.at[sid], src_v)
        pltpu.sync_copy(dst_h.at[sid], dst_v)
        pltpu.sync_copy(z_h, b0)
        r0 = sid * RPT
        for k in range(RPT // CB):
            pltpu.sync_copy(b0, acc.at[pl.ds(r0 + k * CB, CB)])

        @pl.when(cid == 1)
        def _():
            pltpu.sync_copy(z_h.at[:, pl.ds(0, CW)], ones_v)
            for k in range(RPT // CB):
                pltpu.sync_copy(ones_v, cnt_sh.at[pl.ds(r0 + k * CB, CB)])
            pltpu.sync_copy(ones_h, ones_v)
        plsc.subcore_barrier()

        def pipeline(table_h, with_cnt):
            for b in range(P):
                pltpu.async_copy(table_h.at[src_v.at[b]], bufs[b], sems[b])

            @pl.loop(0, CHUNKS - P, step=P)
            def _(j0):
                for b in range(P):
                    pltpu.make_async_copy(
                        table_h.at[src_v.at[b]], bufs[b], sems[b]).wait()
                    pltpu.sync_copy(bufs[b], acc.at[dst_v.at[j0 + b]],
                                    add=True)
                    if with_cnt:
                        pltpu.sync_copy(ones_v, cnt_sh.at[dst_v.at[j0 + b]],
                                        add=True)
                    pltpu.async_copy(
                        table_h.at[src_v.at[j0 + b + P]], bufs[b], sems[b])

            for b in range(P):
                pltpu.make_async_copy(
                    table_h.at[src_v.at[b]], bufs[b], sems[b]).wait()
                pltpu.sync_copy(
                    bufs[b], acc.at[dst_v.at[(CHUNKS - P) + b]], add=True)
                if with_cnt:
                    pltpu.sync_copy(
                        ones_v, cnt_sh.at[dst_v.at[(CHUNKS - P) + b]],
                        add=True)
            plsc.subcore_barrier()

        @pl.when(cid == 0)
        def _():
            pipeline(tlo_h, False)
            pltpu.sync_copy(acc.at[pl.ds(r0, RPT)], outlo_h.at[pl.ds(r0, RPT)])

        @pl.when(cid == 1)
        def _():
            pipeline(thi_h, True)
            pltpu.sync_copy(acc.at[pl.ds(r0, RPT)], outhi_h.at[pl.ds(r0, RPT)])
            pltpu.sync_copy(cnt_sh.at[pl.ds(r0, RPT)], cnt_h.at[pl.ds(r0, RPT)])

    return body(t_lo, t_hi, srcv, dstv, z, ones16)


def _dense1(acc_lo, acc_hi, cnt, x, w1_lT, w1_rT, w_skipT, b1_l, b_skip):
    RB = 1000

    def body(alo_ref, ahi_ref, c_ref, x_ref, wl_ref, wr_ref, ws_ref,
             bl_ref, bs_ref, helo_ref, hehi_ref):
        a = jnp.concatenate([alo_ref[...], ahi_ref[...]], axis=1)
        mean = a / jnp.maximum(c_ref[:, :1], 1.0)
        xb = x_ref[...]
        out1 = (jnp.dot(mean, wl_ref[...], preferred_element_type=jnp.float32)
                + bl_ref[...]
                + jnp.dot(xb, wr_ref[...], preferred_element_type=jnp.float32))
        nrm = jnp.sqrt(jnp.sum(out1 * out1, axis=1, keepdims=True))
        h = out1 / jnp.maximum(nrm, 1e-12)
        h = jnp.tanh(
            h + jnp.dot(xb, ws_ref[...], preferred_element_type=jnp.float32)
            + bs_ref[...])
        helo_ref[...] = h[:, :DH]
        hehi_ref[...] = h[:, DH:]

    return pl.pallas_call(
        body,
        grid=(N // RB,),
        in_specs=[
            pl.BlockSpec((RB, DH), lambda i: (i, 0)),
            pl.BlockSpec((RB, DH), lambda i: (i, 0)),
            pl.BlockSpec((RB, CW), lambda i: (i, 0)),
            pl.BlockSpec((RB, D), lambda i: (i, 0)),
            pl.BlockSpec((D, D), lambda i: (0, 0)),
            pl.BlockSpec((D, D), lambda i: (0, 0)),
            pl.BlockSpec((D, D), lambda i: (0, 0)),
            pl.BlockSpec((1, D), lambda i: (0, 0)),
            pl.BlockSpec((1, D), lambda i: (0, 0)),
        ],
        out_specs=[pl.BlockSpec((RB, DH), lambda i: (i, 0)),
                   pl.BlockSpec((RB, DH), lambda i: (i, 0))],
        out_shape=[jax.ShapeDtypeStruct((N, DH), jnp.float32),
                   jax.ShapeDtypeStruct((N, DH), jnp.float32)],
    )(acc_lo, acc_hi, cnt, x, w1_lT, w1_rT, w_skipT, b1_l, b_skip)


def _dense2(acc_lo, acc_hi, cnt, he_lo, he_hi, w2_lT, w2_rT, w_linT,
            b2_l, b_lin):
    RB = 2000

    def body(alo_ref, ahi_ref, c_ref, hlo_ref, hhi_ref, wl_ref, wr_ref,
             wo_ref, bl_ref, bo_ref, out_ref):
        a = jnp.concatenate([alo_ref[...], ahi_ref[...]], axis=1)
        mean = a / jnp.maximum(c_ref[:, :1], 1.0)
        hb = jnp.concatenate([hlo_ref[...], hhi_ref[...]], axis=1)
        out2 = (jnp.dot(mean, wl_ref[...], preferred_element_type=jnp.float32)
                + bl_ref[...]
                + jnp.dot(hb, wr_ref[...], preferred_element_type=jnp.float32))
        nrm = jnp.sqrt(jnp.sum(out2 * out2, axis=1, keepdims=True))
        h2 = jnp.tanh(out2 / jnp.maximum(nrm, 1e-12))
        out_ref[...] = (
            jnp.dot(h2, wo_ref[...], preferred_element_type=jnp.float32)
            + bo_ref[...])

    return pl.pallas_call(
        body,
        grid=(N // RB,),
        in_specs=[
            pl.BlockSpec((RB, DH), lambda i: (i, 0)),
            pl.BlockSpec((RB, DH), lambda i: (i, 0)),
            pl.BlockSpec((RB, CW), lambda i: (i, 0)),
            pl.BlockSpec((RB, DH), lambda i: (i, 0)),
            pl.BlockSpec((RB, DH), lambda i: (i, 0)),
            pl.BlockSpec((D, D), lambda i: (0, 0)),
            pl.BlockSpec((D, D), lambda i: (0, 0)),
            pl.BlockSpec((D, D), lambda i: (0, 0)),
            pl.BlockSpec((1, D), lambda i: (0, 0)),
            pl.BlockSpec((1, D), lambda i: (0, 0)),
        ],
        out_specs=pl.BlockSpec((RB, D), lambda i: (i, 0)),
        out_shape=jax.ShapeDtypeStruct((N, D), jnp.float32),
    )(acc_lo, acc_hi, cnt, he_lo, he_hi, w2_lT, w2_rT, w_linT, b2_l, b_lin)


def kernel(x, edge_index, w1_l, b1_l, w1_r, w_skip, b_skip,
           w2_l, b2_l, w2_r, w_lin, b_lin):
    src = edge_index[0].astype(jnp.int32)
    dst = edge_index[1].astype(jnp.int32)
    srcv = src.reshape(NS, CHUNKS, CB)
    dst1v = dst.reshape(NS, CHUNKS, CB)
    keep = jnp.asarray(_keep_mask())
    trash_e = jnp.asarray(N + (np.arange(E) % (NROW - N)), jnp.int32)
    dst2v = jnp.where(keep, dst, trash_e).reshape(NS, CHUNKS, CB)

    xe_lo = x[:, :DH]
    xe_hi = x[:, DH:]
    z = jnp.zeros((CB, DH), jnp.float32)
    ones16 = jnp.zeros((CB, CW), jnp.float32).at[:, 0].set(1.0)

    acc1_lo, acc1_hi, cnt1 = _seg_sum(xe_lo, xe_hi, srcv, dst1v, z, ones16)
    he_lo, he_hi = _dense1(acc1_lo, acc1_hi, cnt1, x,
                           w1_l.T, w1_r.T, w_skip.T,
                           b1_l[None, :], b_skip[None, :])
    acc2_lo, acc2_hi, cnt2 = _seg_sum(he_lo, he_hi, srcv, dst2v, z, ones16)
    return _dense2(acc2_lo, acc2_hi, cnt2, he_lo, he_hi,
                   w2_l.T, w2_r.T, w_lin.T, b2_l[None, :], b_lin[None, :])


# speedup vs baseline: 1.1170x; 1.0160x over previous
import functools

import jax
import jax.numpy as jnp
import numpy as np
from jax import lax
from jax.experimental import pallas as pl
from jax.experimental.pallas import tpu as pltpu
from jax.experimental.pallas import tpu_sc as plsc

N = 10000
E = 320000
D = 128
DH = 64
CW = 16
NROW = 10240
NC = 2
NS = 16
CB = 80
CHUNKS = 250
P = 5
RPT = NROW // NS

_KEEP_CACHE = []


def _np_threefry2x32(k1, k2, x1, x2):
    rot = (13, 15, 26, 6, 17, 29, 16, 24)

    def rotl(x, d):
        return ((x << np.uint32(d)) | (x >> np.uint32(32 - d))).astype(np.uint32)

    ks = [np.uint32(k1), np.uint32(k2), np.uint32(k1 ^ k2 ^ 0x1BD11BDA)]
    x = [x1.astype(np.uint32) + ks[0], x2.astype(np.uint32) + ks[1]]
    with np.errstate(over="ignore"):
        for i in range(5):
            for r in (rot[:4], rot[4:])[i % 2]:
                x[0] = (x[0] + x[1]).astype(np.uint32)
                x[1] = x[0] ^ rotl(x[1], r)
            x[0] = (x[0] + ks[(i + 1) % 3]).astype(np.uint32)
            x[1] = (x[1] + ks[(i + 2) % 3] + np.uint32(i + 1)).astype(np.uint32)
    return x


def _keep_mask():
    if not _KEEP_CACHE:
        b1, b2 = _np_threefry2x32(0, 42, np.zeros(E, np.uint32),
                                  np.arange(E, dtype=np.uint32))
        bits = b1 ^ b2
        u = ((bits >> np.uint32(9)) | np.uint32(0x3F800000)).view(np.float32)
        u = np.maximum(np.float32(0.0), u - np.float32(1.0))
        _KEEP_CACHE.append(u >= np.float32(0.3))
    return _KEEP_CACHE[0]


def _seg_sum(t_lo, t_hi, srcv, dstv, z, ones16):
    mesh = plsc.VectorSubcoreMesh(
        core_axis_name="c", subcore_axis_name="s", num_cores=NC, num_subcores=NS
    )
    scratch = (
        [pltpu.VMEM((CHUNKS, CB), jnp.int32)] * 2
        + [pltpu.VMEM((CB, DH), jnp.float32)] * P
        + [pltpu.VMEM((CB, CW), jnp.float32)]
        + [pltpu.VMEM_SHARED((NROW, DH), jnp.float32)]
        + [pltpu.VMEM_SHARED((NROW, CW), jnp.float32)]
        + [pltpu.SemaphoreType.DMA] * P
    )

    @functools.partial(
        pl.kernel,
        mesh=mesh,
        out_type=[jax.ShapeDtypeStruct((NROW, DH), jnp.float32),
                  jax.ShapeDtypeStruct((NROW, DH), jnp.float32),
                  jax.ShapeDtypeStruct((NROW, CW), jnp.float32),
                  jax.ShapeDtypeStruct((NROW, CW), jnp.float32)],
        scratch_types=scratch,
        compiler_params=pltpu.CompilerParams(use_tc_tiling_on_sc=False),
    )
    def body(tlo_h, thi_h, src_h, dst_h, z_h, ones_h,
             outlo_h, outhi_h, cnt0_h, cnt1_h,
             src_v, dst_v, b0, b1, b2, b3, b4, ones_v, acc, cnt_sh,
             s0, s1, s2, s3, s4):
        bufs = [b0, b1, b2, b3, b4]
        sems = [s0, s1, s2, s3, s4]
        cid = lax.axis_index("c")
        sid = lax.axis_index("s")
        pltpu.sync_copy(src_h.at[sid], src_v)
        pltpu.sync_copy(dst_h.at[sid], dst_v)
        pltpu.sync_copy(z_h, b0)
        r0 = sid * RPT
        for k in range(RPT // CB):
            pltpu.sync_copy(b0, acc.at[pl.ds(r0 + k * CB, CB)])

        pltpu.sync_copy(z_h.at[:, pl.ds(0, CW)], ones_v)
        for k in range(RPT // CB):
            pltpu.sync_copy(ones_v, cnt_sh.at[pl.ds(r0 + k * CB, CB)])
        pltpu.sync_copy(ones_h, ones_v)
        plsc.subcore_barrier()

        def pipeline(table_h, parity):
            for b in range(P):
                pltpu.async_copy(table_h.at[src_v.at[b]], bufs[b], sems[b])

            @pl.loop(0, CHUNKS - P, step=P)
            def _(j0):
                for b in range(P):
                    pltpu.make_async_copy(
                        table_h.at[src_v.at[b]], bufs[b], sems[b]).wait()
                    pltpu.sync_copy(bufs[b], acc.at[dst_v.at[j0 + b]],
                                    add=True)
                    if b % 2 == parity:
                        pltpu.sync_copy(ones_v, cnt_sh.at[dst_v.at[j0 + b]],
                                        add=True)
                    pltpu.async_copy(
                        table_h.at[src_v.at[j0 + b + P]], bufs[b], sems[b])

            for b in range(P):
                pltpu.make_async_copy(
                    table_h.at[src_v.at[b]], bufs[b], sems[b]).wait()
                pltpu.sync_copy(
                    bufs[b], acc.at[dst_v.at[(CHUNKS - P) + b]], add=True)
                if b % 2 == parity:
                    pltpu.sync_copy(
                        ones_v, cnt_sh.at[dst_v.at[(CHUNKS - P) + b]],
                        add=True)
            plsc.subcore_barrier()

        @pl.when(cid == 0)
        def _():
            pipeline(tlo_h, 0)
            pltpu.sync_copy(acc.at[pl.ds(r0, RPT)], outlo_h.at[pl.ds(r0, RPT)])
            pltpu.sync_copy(cnt_sh.at[pl.ds(r0, RPT)],
                            cnt0_h.at[pl.ds(r0, RPT)])

        @pl.when(cid == 1)
        def _():
            pipeline(thi_h, 1)
            pltpu.sync_copy(acc.at[pl.ds(r0, RPT)], outhi_h.at[pl.ds(r0, RPT)])
            pltpu.sync_copy(cnt_sh.at[pl.ds(r0, RPT)],
                            cnt1_h.at[pl.ds(r0, RPT)])

    return body(t_lo, t_hi, srcv, dstv, z, ones16)


def _dense1(acc_lo, acc_hi, cnt0, cnt1, x, w1_lT, w1_rT, w_skipT,
            b1_l, b_skip):
    RB = 1000

    def body(alo_ref, ahi_ref, c0_ref, c1_ref, x_ref, wl_ref, wr_ref,
             ws_ref, bl_ref, bs_ref, helo_ref, hehi_ref):
        a = jnp.concatenate([alo_ref[...], ahi_ref[...]], axis=1)
        c = c0_ref[:, :1] + c1_ref[:, :1]
        mean = a / jnp.maximum(c, 1.0)
        xb = x_ref[...]
        out1 = (jnp.dot(mean, wl_ref[...], preferred_element_type=jnp.float32)
                + bl_ref[...]
                + jnp.dot(xb, wr_ref[...], preferred_element_type=jnp.float32))
        nrm = jnp.sqrt(jnp.sum(out1 * out1, axis=1, keepdims=True))
        h = out1 / jnp.maximum(nrm, 1e-12)
        h = jnp.tanh(
            h + jnp.dot(xb, ws_ref[...], preferred_element_type=jnp.float32)
            + bs_ref[...])
        helo_ref[...] = h[:, :DH]
        hehi_ref[...] = h[:, DH:]

    return pl.pallas_call(
        body,
        grid=(N // RB,),
        in_specs=[
            pl.BlockSpec((RB, DH), lambda i: (i, 0)),
            pl.BlockSpec((RB, DH), lambda i: (i, 0)),
            pl.BlockSpec((RB, CW), lambda i: (i, 0)),
            pl.BlockSpec((RB, CW), lambda i: (i, 0)),
            pl.BlockSpec((RB, D), lambda i: (i, 0)),
            pl.BlockSpec((D, D), lambda i: (0, 0)),
            pl.BlockSpec((D, D), lambda i: (0, 0)),
            pl.BlockSpec((D, D), lambda i: (0, 0)),
            pl.BlockSpec((1, D), lambda i: (0, 0)),
            pl.BlockSpec((1, D), lambda i: (0, 0)),
        ],
        out_specs=[pl.BlockSpec((RB, DH), lambda i: (i, 0)),
                   pl.BlockSpec((RB, DH), lambda i: (i, 0))],
        out_shape=[jax.ShapeDtypeStruct((N, DH), jnp.float32),
                   jax.ShapeDtypeStruct((N, DH), jnp.float32)],
    )(acc_lo, acc_hi, cnt0, cnt1, x, w1_lT, w1_rT, w_skipT, b1_l, b_skip)


def _dense2(acc_lo, acc_hi, cnt0, cnt1, he_lo, he_hi, w2_lT, w2_rT,
            w_linT, b2_l, b_lin):
    RB = 2000

    def body(alo_ref, ahi_ref, c0_ref, c1_ref, hlo_ref, hhi_ref, wl_ref,
             wr_ref, wo_ref, bl_ref, bo_ref, out_ref):
        a = jnp.concatenate([alo_ref[...], ahi_ref[...]], axis=1)
        c = c0_ref[:, :1] + c1_ref[:, :1]
        mean = a / jnp.maximum(c, 1.0)
        hb = jnp.concatenate([hlo_ref[...], hhi_ref[...]], axis=1)
        out2 = (jnp.dot(mean, wl_ref[...], preferred_element_type=jnp.float32)
                + bl_ref[...]
                + jnp.dot(hb, wr_ref[...], preferred_element_type=jnp.float32))
        nrm = jnp.sqrt(jnp.sum(out2 * out2, axis=1, keepdims=True))
        h2 = jnp.tanh(out2 / jnp.maximum(nrm, 1e-12))
        out_ref[...] = (
            jnp.dot(h2, wo_ref[...], preferred_element_type=jnp.float32)
            + bo_ref[...])

    return pl.pallas_call(
        body,
        grid=(N // RB,),
        in_specs=[
            pl.BlockSpec((RB, DH), lambda i: (i, 0)),
            pl.BlockSpec((RB, DH), lambda i: (i, 0)),
            pl.BlockSpec((RB, CW), lambda i: (i, 0)),
            pl.BlockSpec((RB, CW), lambda i: (i, 0)),
            pl.BlockSpec((RB, DH), lambda i: (i, 0)),
            pl.BlockSpec((RB, DH), lambda i: (i, 0)),
            pl.BlockSpec((D, D), lambda i: (0, 0)),
            pl.BlockSpec((D, D), lambda i: (0, 0)),
            pl.BlockSpec((D, D), lambda i: (0, 0)),
            pl.BlockSpec((1, D), lambda i: (0, 0)),
            pl.BlockSpec((1, D), lambda i: (0, 0)),
        ],
        out_specs=pl.BlockSpec((RB, D), lambda i: (i, 0)),
        out_shape=jax.ShapeDtypeStruct((N, D), jnp.float32),
    )(acc_lo, acc_hi, cnt0, cnt1, he_lo, he_hi, w2_lT, w2_rT, w_linT,
      b2_l, b_lin)


def kernel(x, edge_index, w1_l, b1_l, w1_r, w_skip, b_skip,
           w2_l, b2_l, w2_r, w_lin, b_lin):
    src = edge_index[0].astype(jnp.int32)
    dst = edge_index[1].astype(jnp.int32)
    srcv = src.reshape(NS, CHUNKS, CB)
    dst1v = dst.reshape(NS, CHUNKS, CB)
    keep = jnp.asarray(_keep_mask())
    trash_e = jnp.asarray(N + (np.arange(E) % (NROW - N)), jnp.int32)
    dst2v = jnp.where(keep, dst, trash_e).reshape(NS, CHUNKS, CB)

    xe_lo = x[:, :DH]
    xe_hi = x[:, DH:]
    z = jnp.zeros((CB, DH), jnp.float32)
    ones16 = jnp.zeros((CB, CW), jnp.float32).at[:, 0].set(1.0)

    acc1_lo, acc1_hi, c1a, c1b = _seg_sum(xe_lo, xe_hi, srcv, dst1v, z,
                                          ones16)
    he_lo, he_hi = _dense1(acc1_lo, acc1_hi, c1a, c1b, x,
                           w1_l.T, w1_r.T, w_skip.T,
                           b1_l[None, :], b_skip[None, :])
    acc2_lo, acc2_hi, c2a, c2b = _seg_sum(he_lo, he_hi, srcv, dst2v, z,
                                          ones16)
    return _dense2(acc2_lo, acc2_hi, c2a, c2b, he_lo, he_hi,
                   w2_l.T, w2_r.T, w_lin.T, b2_l[None, :], b_lin[None, :])
